# Initial kernel scaffold; baseline (speedup 1.0000x reference)
import jax
import jax.numpy as jnp
from jax.experimental import pallas as pl


def kernel(latents, policy_indices, actions, W1, b1, W2, b2):
    raise NotImplementedError("write your pallas kernel here")


# speedup vs baseline: 1.5464x; 1.5464x over previous
import functools

import jax
import jax.numpy as jnp
from jax import lax
from jax.experimental import pallas as pl
from jax.experimental.pallas import tpu as pltpu
from jax.experimental.pallas import tpu_sc as plsc

NPOL = 8
DM = 1024
DF = 4096
AD = 32
APAD = 128
NT = 8192

BT = 512
NB = NT // BT + NPOL
PADN = NB * BT
FB = 512
NFB = DF // FB
TRASH = NT

NW = 32
RPW = PADN // NW
CH = 64
NCH = RPW // CH

@functools.cache
def _build_gather_sc():
    mesh = plsc.VectorSubcoreMesh(core_axis_name="c", subcore_axis_name="s")

    @functools.partial(
        pl.kernel,
        mesh=mesh,
        out_type=(
            jax.ShapeDtypeStruct((PADN, DM), jnp.float32),
            jax.ShapeDtypeStruct((PADN, APAD), jnp.float32),
        ),
        scratch_types=[
            pltpu.VMEM((CH,), jnp.int32),
            pltpu.VMEM((CH, DM), jnp.float32),
            pltpu.VMEM((CH, APAD), jnp.float32),
            pltpu.SemaphoreType.DMA,
            pltpu.SemaphoreType.DMA,
        ],
    )
    def gather_k(z_hbm, a_hbm, src_hbm, oz_hbm, oa_hbm,
                 idx_v, zrows_v, arows_v, semz, sema):
        wid = lax.axis_index("s") * 2 + lax.axis_index("c")
        base = wid * RPW
        for c in range(NCH):
            off = base + c * CH
            pltpu.sync_copy(src_hbm.at[pl.ds(off, CH)], idx_v)
            cz = pltpu.async_copy(z_hbm.at[idx_v], zrows_v, semz)
            ca = pltpu.async_copy(a_hbm.at[idx_v], arows_v, sema)
            cz.wait()
            ca.wait()
            pltpu.sync_copy(zrows_v, oz_hbm.at[pl.ds(off, CH)])
            pltpu.sync_copy(arows_v, oa_hbm.at[pl.ds(off, CH)])

    return gather_k


@functools.cache
def _build_scatter_sc():
    mesh = plsc.VectorSubcoreMesh(core_axis_name="c", subcore_axis_name="s")

    @functools.partial(
        pl.kernel,
        mesh=mesh,
        out_type=jax.ShapeDtypeStruct((NT + 8, DM), jnp.float32),
        scratch_types=[
            pltpu.VMEM((CH,), jnp.int32),
            pltpu.VMEM((CH, DM), jnp.float32),
            pltpu.SemaphoreType.DMA,
        ],
    )
    def scatter_k(ys_hbm, dst_hbm, out_hbm, idx_v, rows_v, sem):
        wid = lax.axis_index("s") * 2 + lax.axis_index("c")
        base = wid * RPW
        for c in range(NCH):
            off = base + c * CH
            pltpu.sync_copy(dst_hbm.at[pl.ds(off, CH)], idx_v)
            pltpu.sync_copy(ys_hbm.at[pl.ds(off, CH)], rows_v)
            pltpu.async_copy(rows_v, out_hbm.at[idx_v], sem).wait()

    return scatter_k


def _gather_sc(latents, actions, src):
    return _build_gather_sc()(latents, actions, src)


def _scatter_sc(ys, dst):
    return _build_scatter_sc()(ys, dst)


def _mlp_body(be_ref, xz_ref, xa_ref, w1z_ref, w1a_ref, b1_ref, w2_ref,
              b2_ref, o_ref):
    j = pl.program_id(1)
    h = lax.dot_general(xz_ref[...], w1z_ref[0], (((1,), (0,)), ((), ())),
                        preferred_element_type=jnp.float32)
    w1a = jnp.concatenate(
        [w1a_ref[0], jnp.zeros((APAD - AD, FB), jnp.float32)], axis=0)
    h = h + lax.dot_general(xa_ref[...], w1a, (((1,), (0,)), ((), ())),
                            preferred_element_type=jnp.float32)
    h = jnp.maximum(h + b1_ref[0], 0.0)
    y = lax.dot_general(h, w2_ref[0], (((1,), (0,)), ((), ())),
                        preferred_element_type=jnp.float32)

    @pl.when(j == 0)
    def _():
        o_ref[...] = y + b2_ref[0]

    @pl.when(j != 0)
    def _():
        o_ref[...] = o_ref[...] + y


def _mlp_grid_spec():
    return pltpu.PrefetchScalarGridSpec(
        num_scalar_prefetch=1,
        grid=(NB, NFB),
        in_specs=[
            pl.BlockSpec((BT, DM), lambda i, j, be: (i, 0)),
            pl.BlockSpec((BT, APAD), lambda i, j, be: (i, 0)),
            pl.BlockSpec((1, DM, FB), lambda i, j, be: (be[i], 0, j)),
            pl.BlockSpec((1, AD, FB), lambda i, j, be: (be[i], DM // AD, j)),
            pl.BlockSpec((1, 1, FB), lambda i, j, be: (be[i], 0, j)),
            pl.BlockSpec((1, FB, DM), lambda i, j, be: (be[i], j, 0)),
            pl.BlockSpec((1, 1, DM), lambda i, j, be: (be[i], 0, 0)),
        ],
        out_specs=pl.BlockSpec((BT, DM), lambda i, j, be: (i, 0)),
    )


def _grouped_mlp(be, xz, xa, W1, b1, W2, b2):
    return pl.pallas_call(
        _mlp_body,
        grid_spec=_mlp_grid_spec(),
        out_shape=jax.ShapeDtypeStruct((PADN, DM), jnp.float32),
        compiler_params=pltpu.CompilerParams(
            dimension_semantics=("arbitrary", "arbitrary")),
    )(be, xz, xa, W1, W1, b1.reshape(NPOL, 1, DF), W2, b2.reshape(NPOL, 1, DM))


def _route(pid):
    pid = pid.astype(jnp.int32)
    order = jnp.argsort(pid).astype(jnp.int32)
    counts = jnp.bincount(pid, length=NPOL).astype(jnp.int32)
    offs = jnp.concatenate(
        [jnp.zeros((1,), jnp.int32), jnp.cumsum(counts)[:-1].astype(jnp.int32)])
    nbe = (counts + BT - 1) // BT
    cnb = jnp.cumsum(nbe).astype(jnp.int32)
    bid = jnp.arange(NB, dtype=jnp.int32)
    be = jnp.searchsorted(cnb, bid, side="right").astype(jnp.int32)
    be = jnp.minimum(be, NPOL - 1)
    bstart = jnp.concatenate(
        [jnp.zeros((1,), jnp.int32), cnb[:-1]])
    brank = bid - bstart[be]
    p = jnp.arange(PADN, dtype=jnp.int32)
    blk = p // BT
    e = be[blk]
    r = brank[blk] * BT + (p % BT)
    valid = r < counts[e]
    src = jnp.where(valid, order[jnp.clip(offs[e] + r, 0, NT - 1)], 0)
    dst = jnp.where(valid, src, TRASH).astype(jnp.int32)
    return be, src.astype(jnp.int32), dst


def kernel(latents, policy_indices, actions, W1, b1, W2, b2):
    be, src, dst = _route(policy_indices)
    ap = jnp.pad(actions, ((0, 0), (0, APAD - AD)))
    xz, xa = _gather_sc(latents, ap, src)
    ys = _grouped_mlp(be, xz, xa, W1, b1, W2, b2)
    out = _scatter_sc(ys, dst)
    return out[:NT]


# speedup vs baseline: 1.5474x; 1.0006x over previous
import functools

import jax
import jax.numpy as jnp
from jax import lax
from jax.experimental import pallas as pl
from jax.experimental.pallas import tpu as pltpu
from jax.experimental.pallas import tpu_sc as plsc

NPOL = 8
DM = 1024
DF = 4096
AD = 32
APAD = 128
NT = 8192

BT = 512
NB = NT // BT + NPOL
PADN = NB * BT
FB = 512
NFB = DF // FB
TRASH = NT

NW = 32
RPW = PADN // NW
CH = 64
NCH = RPW // CH

@functools.cache
def _build_gather_sc():
    mesh = plsc.VectorSubcoreMesh(core_axis_name="c", subcore_axis_name="s")

    @functools.partial(
        pl.kernel,
        mesh=mesh,
        out_type=(
            jax.ShapeDtypeStruct((PADN, DM), jnp.float32),
            jax.ShapeDtypeStruct((PADN, APAD), jnp.float32),
        ),
        scratch_types=[
            pltpu.VMEM((CH,), jnp.int32),
            pltpu.VMEM((CH, DM), jnp.float32),
            pltpu.VMEM((CH, APAD), jnp.float32),
            pltpu.SemaphoreType.DMA,
            pltpu.SemaphoreType.DMA,
        ],
    )
    def gather_k(z_hbm, a_hbm, src_hbm, oz_hbm, oa_hbm,
                 idx_v, zrows_v, arows_v, semz, sema):
        wid = lax.axis_index("s") * 2 + lax.axis_index("c")
        base = wid * RPW
        for c in range(NCH):
            off = base + c * CH
            pltpu.sync_copy(src_hbm.at[pl.ds(off, CH)], idx_v)
            cz = pltpu.async_copy(z_hbm.at[idx_v], zrows_v, semz)
            ca = pltpu.async_copy(a_hbm.at[idx_v], arows_v, sema)
            cz.wait()
            ca.wait()
            pltpu.sync_copy(zrows_v, oz_hbm.at[pl.ds(off, CH)])
            pltpu.sync_copy(arows_v, oa_hbm.at[pl.ds(off, CH)])

    return gather_k


@functools.cache
def _build_scatter_sc():
    mesh = plsc.VectorSubcoreMesh(core_axis_name="c", subcore_axis_name="s")

    @functools.partial(
        pl.kernel,
        mesh=mesh,
        out_type=jax.ShapeDtypeStruct((NT + 8, DM), jnp.float32),
        scratch_types=[
            pltpu.VMEM((CH,), jnp.int32),
            pltpu.VMEM((CH, DM), jnp.float32),
            pltpu.SemaphoreType.DMA,
        ],
    )
    def scatter_k(ys_hbm, dst_hbm, out_hbm, idx_v, rows_v, sem):
        wid = lax.axis_index("s") * 2 + lax.axis_index("c")
        base = wid * RPW
        for c in range(NCH):
            off = base + c * CH
            pltpu.sync_copy(dst_hbm.at[pl.ds(off, CH)], idx_v)
            pltpu.sync_copy(ys_hbm.at[pl.ds(off, CH)], rows_v)
            pltpu.async_copy(rows_v, out_hbm.at[idx_v], sem).wait()

    return scatter_k


def _gather_sc(latents, actions, src):
    return _build_gather_sc()(latents, actions, src)


def _scatter_sc(ys, dst):
    return _build_scatter_sc()(ys, dst)


def _mlp_body(be_ref, xz_ref, xa_ref, w1z_ref, w1a_ref, b1_ref, w2_ref,
              b2_ref, o_ref):
    j = pl.program_id(1)
    bf = jnp.bfloat16
    h = lax.dot_general(xz_ref[...].astype(bf), w1z_ref[0].astype(bf),
                        (((1,), (0,)), ((), ())),
                        preferred_element_type=jnp.float32)
    w1a = jnp.concatenate(
        [w1a_ref[0], jnp.zeros((APAD - AD, FB), jnp.float32)], axis=0)
    h = h + lax.dot_general(xa_ref[...].astype(bf), w1a.astype(bf),
                            (((1,), (0,)), ((), ())),
                            preferred_element_type=jnp.float32)
    h = jnp.maximum(h + b1_ref[0], 0.0)
    y = lax.dot_general(h.astype(bf), w2_ref[0].astype(bf),
                        (((1,), (0,)), ((), ())),
                        preferred_element_type=jnp.float32)

    @pl.when(j == 0)
    def _():
        o_ref[...] = y + b2_ref[0]

    @pl.when(j != 0)
    def _():
        o_ref[...] = o_ref[...] + y


def _mlp_grid_spec():
    return pltpu.PrefetchScalarGridSpec(
        num_scalar_prefetch=1,
        grid=(NB, NFB),
        in_specs=[
            pl.BlockSpec((BT, DM), lambda i, j, be: (i, 0)),
            pl.BlockSpec((BT, APAD), lambda i, j, be: (i, 0)),
            pl.BlockSpec((1, DM, FB), lambda i, j, be: (be[i], 0, j)),
            pl.BlockSpec((1, AD, FB), lambda i, j, be: (be[i], DM // AD, j)),
            pl.BlockSpec((1, 1, FB), lambda i, j, be: (be[i], 0, j)),
            pl.BlockSpec((1, FB, DM), lambda i, j, be: (be[i], j, 0)),
            pl.BlockSpec((1, 1, DM), lambda i, j, be: (be[i], 0, 0)),
        ],
        out_specs=pl.BlockSpec((BT, DM), lambda i, j, be: (i, 0)),
    )


def _grouped_mlp(be, xz, xa, W1, b1, W2, b2):
    return pl.pallas_call(
        _mlp_body,
        grid_spec=_mlp_grid_spec(),
        out_shape=jax.ShapeDtypeStruct((PADN, DM), jnp.float32),
        compiler_params=pltpu.CompilerParams(
            dimension_semantics=("arbitrary", "arbitrary")),
    )(be, xz, xa, W1, W1, b1.reshape(NPOL, 1, DF), W2, b2.reshape(NPOL, 1, DM))


def _route(pid):
    pid = pid.astype(jnp.int32)
    order = jnp.argsort(pid).astype(jnp.int32)
    counts = jnp.bincount(pid, length=NPOL).astype(jnp.int32)
    offs = jnp.concatenate(
        [jnp.zeros((1,), jnp.int32), jnp.cumsum(counts)[:-1].astype(jnp.int32)])
    nbe = (counts + BT - 1) // BT
    cnb = jnp.cumsum(nbe).astype(jnp.int32)
    bid = jnp.arange(NB, dtype=jnp.int32)
    be = jnp.searchsorted(cnb, bid, side="right").astype(jnp.int32)
    be = jnp.minimum(be, NPOL - 1)
    bstart = jnp.concatenate(
        [jnp.zeros((1,), jnp.int32), cnb[:-1]])
    brank = bid - bstart[be]
    p = jnp.arange(PADN, dtype=jnp.int32)
    blk = p // BT
    e = be[blk]
    r = brank[blk] * BT + (p % BT)
    valid = r < counts[e]
    src = jnp.where(valid, order[jnp.clip(offs[e] + r, 0, NT - 1)], 0)
    dst = jnp.where(valid, src, TRASH).astype(jnp.int32)
    return be, src.astype(jnp.int32), dst


def kernel(latents, policy_indices, actions, W1, b1, W2, b2):
    be, src, dst = _route(policy_indices)
    ap = jnp.pad(actions, ((0, 0), (0, APAD - AD)))
    xz, xa = _gather_sc(latents, ap, src)
    ys = _grouped_mlp(be, xz, xa, W1, b1, W2, b2)
    out = _scatter_sc(ys, dst)
    return out[:NT]
